# baseline (speedup 1.0000x reference)
import jax
import jax.numpy as jnp
from jax import lax
from jax.experimental import pallas as pl
from jax.experimental.pallas import tpu as pltpu
from jax.experimental.pallas import tpu_sc as plsc

Q = 1024
DIM = 16
N = 100000
G = 64
NP = 106496
NG = NP // G
L = 16
NGV = NG // L
NSEL = 16
K = 10
PAD_VAL = 1e15
NC, NS = 2, 16
NW = NC * NS
QPW = Q // NW
BIG = 3e38
CMBW = NG + DIM * L

CB = 8192
QB = 512


def _gm_body(yb_ref, xT_ref, gm_ref):
    yb = yb_ref[...]
    xT = xT_ref[...]
    ybh = yb.astype(jnp.bfloat16)
    ybl = (yb - ybh.astype(jnp.float32)).astype(jnp.bfloat16)
    xh = xT.astype(jnp.bfloat16)
    xl = (xT - xh.astype(jnp.float32)).astype(jnp.bfloat16)
    dn = (((1,), (0,)), ((), ()))
    z = (lax.dot_general(ybh, xh, dn, preferred_element_type=jnp.float32)
         + (lax.dot_general(ybh, xl, dn, preferred_element_type=jnp.float32)
            + lax.dot_general(ybl, xh, dn,
                              preferred_element_type=jnp.float32)))
    yn = jnp.sum(yb * yb, axis=1, keepdims=True)
    s = yn - 2.0 * z
    r = jnp.min(s.reshape(CB // G, G, QB), axis=1)
    gm_ref[...] = r.T


def _sc_body(cmb_hbm, yg_hbm, out_hbm,
             cmb_t0, cmb_t1, sel_t0, sel_t1, rows_t0, rows_t1,
             out_t, sem0, sem1, semc):
    cid = lax.axis_index("c")
    sid = lax.axis_index("s")
    w = sid * NC + cid

    iota16 = lax.iota(jnp.int32, 16)
    ioD = iota16 * DIM
    inf16 = jnp.full((L,), BIG, jnp.float32)
    zero16 = jnp.zeros((L,), jnp.int32)

    def bmerge(v, ids, c3):
        td, ti, _ = c3
        sd, si = plsc.sort_key_val(v, ids)
        rd = jnp.flip(sd, 0)
        ri = jnp.flip(si, 0)
        keep = td <= rd
        nd = jnp.where(keep, td, rd)
        ni = jnp.where(keep, ti, ri)
        nd, ni = plsc.sort_key_val(nd, ni)
        return nd, ni, jnp.max(nd)

    def phase_a(cmb_t):
        def stepA(i, c3):
            vs = [cmb_t[pl.ds((i * 4 + u) * L, L)] for u in range(4)]
            ms = [jnp.min(v) for v in vs]
            mc = jnp.minimum(jnp.minimum(ms[0], ms[1]),
                             jnp.minimum(ms[2], ms[3]))

            def do(c):
                for u in range(4):
                    ids = iota16 + (i * 4 + u) * L
                    c = lax.cond(
                        ms[u] < c[2],
                        lambda c2, v=vs[u], d=ids: bmerge(v, d, c2),
                        lambda c2: c2, c)
                return c

            return lax.cond(mc < c3[2], do, lambda c: c, c3)

        _, tiA, _ = lax.fori_loop(
            0, NGV // 4, stepA, (inf16, zero16, jnp.float32(BIG)))
        return tiA

    def phase_b(cmb_t, sel_t, rows_t):
        def stepB(s_, c3):
            rowsplat = jnp.full((L,), s_, jnp.int32)
            d2s, ms = [], []
            for v_ in range(4):
                colbase = ioD + v_ * (L * DIM)
                sq = []
                for j in range(DIM):
                    yv = plsc.load_gather(rows_t, [rowsplat, colbase + j])
                    d = yv - cmb_t[pl.ds(NG + j * L, L)]
                    sq.append(d * d)
                a = [sq[i] + sq[i + 8] for i in range(8)]
                b = [a[i] + a[i + 4] for i in range(4)]
                c2 = [b[i] + b[i + 2] for i in range(2)]
                d2s.append(c2[0] + c2[1])
                ms.append(jnp.min(d2s[v_]))
            mc = jnp.minimum(jnp.minimum(ms[0], ms[1]),
                             jnp.minimum(ms[2], ms[3]))

            def do(c):
                gid = plsc.load_gather(sel_t, [rowsplat])
                for v_ in range(4):
                    cidx = gid * G + v_ * L + iota16
                    c = lax.cond(
                        ms[v_] < c[2],
                        lambda c2, v=d2s[v_], d=cidx: bmerge(v, d, c2),
                        lambda c2: c2, c)
                return c

            return lax.cond(mc < c3[2], do, lambda c: c, c3)

        btd, bti, _ = lax.fori_loop(
            0, NSEL, stepB, (inf16, zero16, jnp.float32(BIG)))
        return btd, bti

    def emit(q, btd, bti):
        u = plsc.bitcast(btd, jnp.int32)
        y0 = plsc.bitcast((u >> 1) + 0x1FBD1DF6, jnp.float32)
        y1 = 0.5 * (y0 + btd / y0)
        y2 = 0.5 * (y1 + btd / y1)
        y3 = 0.5 * (y2 + btd / y2)
        out_t[pl.ds(0, L)] = plsc.bitcast(y3, jnp.int32)
        out_t[pl.ds(L, L)] = bti
        pltpu.sync_copy(out_t, out_hbm.at[pl.ds(q * 2 * L, 2 * L)])

    def per_pair(p, carry):
        q0 = w * QPW + 2 * p
        q1 = q0 + 1
        pltpu.sync_copy(cmb_hbm.at[pl.ds(q0 * CMBW, CMBW)], cmb_t0)
        sel_t0[...] = phase_a(cmb_t0)
        cp0 = pltpu.async_copy(yg_hbm.at[sel_t0], rows_t0, sem0)
        pltpu.sync_copy(cmb_hbm.at[pl.ds(q1 * CMBW, CMBW)], cmb_t1)
        sel_t1[...] = phase_a(cmb_t1)
        cp1 = pltpu.async_copy(yg_hbm.at[sel_t1], rows_t1, sem1)
        cp0.wait()
        btd, bti = phase_b(cmb_t0, sel_t0, rows_t0)
        emit(q0, btd, bti)
        cp1.wait()
        btd, bti = phase_b(cmb_t1, sel_t1, rows_t1)
        emit(q1, btd, bti)
        return carry

    lax.fori_loop(0, QPW // 2, per_pair, 0)


def kernel(x, y, k, n_splits):
    del k, n_splits
    yp = jnp.concatenate(
        [y, jnp.full((NP - N, DIM), PAD_VAL, jnp.float32)], axis=0)
    gm = pl.pallas_call(
        _gm_body,
        grid=(NP // CB, Q // QB),
        in_specs=[
            pl.BlockSpec((CB, DIM), lambda i, j: (i, 0)),
            pl.BlockSpec((DIM, QB), lambda i, j: (0, j)),
        ],
        out_specs=pl.BlockSpec((QB, CB // G), lambda i, j: (j, i)),
        out_shape=jax.ShapeDtypeStruct((Q, NG), jnp.float32),
    )(yp, x.T)
    yg = yp.reshape(NG, G * DIM)
    xbB = jnp.broadcast_to(x.reshape(Q * DIM, 1), (Q * DIM, L))
    cmb = jnp.concatenate([gm, xbB.reshape(Q, DIM * L)], axis=1)

    sc_call = pl.kernel(
        _sc_body,
        out_type=jax.ShapeDtypeStruct((Q * 2 * L,), jnp.int32),
        mesh=plsc.VectorSubcoreMesh(core_axis_name="c", subcore_axis_name="s"),
        compiler_params=pltpu.CompilerParams(needs_layout_passes=False),
        scratch_types=[
            pltpu.VMEM((CMBW,), jnp.float32),
            pltpu.VMEM((CMBW,), jnp.float32),
            pltpu.VMEM((NSEL,), jnp.int32),
            pltpu.VMEM((NSEL,), jnp.int32),
            pltpu.VMEM((NSEL, G * DIM), jnp.float32),
            pltpu.VMEM((NSEL, G * DIM), jnp.float32),
            pltpu.VMEM((2 * L,), jnp.int32),
            pltpu.SemaphoreType.DMA,
            pltpu.SemaphoreType.DMA,
            pltpu.SemaphoreType.DMA,
        ],
    )
    out = sc_call(cmb.reshape(-1), yg).reshape(Q, 2 * L)
    ds = lax.bitcast_convert_type(out[:, :K], jnp.float32)
    di = out[:, L:L + K]
    return ds, di[..., None]


# speedup vs baseline: 1.1032x; 1.1032x over previous
import jax
import jax.numpy as jnp
from jax import lax
from jax.experimental import pallas as pl
from jax.experimental.pallas import tpu as pltpu
from jax.experimental.pallas import tpu_sc as plsc

Q = 1024
DIM = 16
N = 100000
G = 64
NP = 106496
NG = NP // G
L = 16
NGV = NG // L
NSEL = 16
K = 10
PAD_VAL = 1e15
NC, NS = 2, 16
NW = NC * NS
QPW = Q // NW
BIG = 3e38
CMBW = NG + DIM * L

CB = 8192
QB = 512


def _gm_body(yb_ref, xT_ref, gm_ref):
    yb = yb_ref[...]
    xT = xT_ref[...]
    ybh = yb.astype(jnp.bfloat16)
    ybl = (yb - ybh.astype(jnp.float32)).astype(jnp.bfloat16)
    xh = xT.astype(jnp.bfloat16)
    xl = (xT - xh.astype(jnp.float32)).astype(jnp.bfloat16)
    dn = (((1,), (0,)), ((), ()))
    z = (lax.dot_general(ybh, xh, dn, preferred_element_type=jnp.float32)
         + (lax.dot_general(ybh, xl, dn, preferred_element_type=jnp.float32)
            + lax.dot_general(ybl, xh, dn,
                              preferred_element_type=jnp.float32)))
    yn = jnp.sum(yb * yb, axis=1, keepdims=True)
    s = yn - 2.0 * z
    r = jnp.min(s.reshape(CB // G, G, QB), axis=1)
    gm_ref[...] = r.T


NQI = 4


def _sc_body(cmb_hbm, yg_hbm, out_hbm,
             cmb_ts, sel_ts, rows_ts, out_t, gsems, csems):
    cid = lax.axis_index("c")
    sid = lax.axis_index("s")
    w = sid * NC + cid

    iota16 = lax.iota(jnp.int32, 16)
    ioD = iota16 * DIM
    inf16 = jnp.full((L,), BIG, jnp.float32)
    zero16 = jnp.zeros((L,), jnp.int32)

    def phase_a(cmb_t):
        def stepA(i, c3):
            td, ti, kth = c3
            v = cmb_t[pl.ds(i * L, L)]
            m = jnp.min(v)

            def merge(_):
                ids = iota16 + i * L
                sd, si = plsc.sort_key_val(v, ids)
                rd = jnp.flip(sd, 0)
                ri = jnp.flip(si, 0)
                keep = td <= rd
                nd = jnp.where(keep, td, rd)
                ni = jnp.where(keep, ti, ri)
                nd, ni = plsc.sort_key_val(nd, ni)
                return nd, ni, jnp.max(nd)

            return lax.cond(m < kth, merge, lambda _: c3, None)

        _, tiA, _ = lax.fori_loop(
            0, NGV, stepA, (inf16, zero16, jnp.float32(BIG)))
        return tiA

    def phase_b(cmb_t, sel_t, rows_t):
        def stepB(t, c3):
            btd, bti, kth = c3
            s_ = t // 4
            v_ = t % 4
            rowsplat = jnp.full((L,), s_, jnp.int32)
            colbase = ioD + v_ * (L * DIM)
            sq = []
            for j in range(DIM):
                yv = plsc.load_gather(rows_t, [rowsplat, colbase + j])
                d = yv - cmb_t[pl.ds(NG + j * L, L)]
                sq.append(d * d)
            a = [sq[i] + sq[i + 8] for i in range(8)]
            b = [a[i] + a[i + 4] for i in range(4)]
            c2 = [b[i] + b[i + 2] for i in range(2)]
            d2v = c2[0] + c2[1]
            m = jnp.min(d2v)

            def merge(_):
                gid = plsc.load_gather(sel_t, [rowsplat])
                cidx = gid * G + v_ * L + iota16
                sd, si = plsc.sort_key_val(d2v, cidx)
                rd = jnp.flip(sd, 0)
                ri = jnp.flip(si, 0)
                keep = btd <= rd
                nd = jnp.where(keep, btd, rd)
                ni = jnp.where(keep, bti, ri)
                nd, ni = plsc.sort_key_val(nd, ni)
                return nd, ni, jnp.max(nd)

            return lax.cond(m < kth, merge, lambda _: c3, None)

        btd, bti, _ = lax.fori_loop(
            0, NSEL * (G // L), stepB, (inf16, zero16, jnp.float32(BIG)))
        return btd, bti

    def emit(u, btd, bti):
        iv = plsc.bitcast(btd, jnp.int32)
        y0 = plsc.bitcast((iv >> 1) + 0x1FBD1DF6, jnp.float32)
        y1 = 0.5 * (y0 + btd / y0)
        y2 = 0.5 * (y1 + btd / y1)
        y3 = 0.5 * (y2 + btd / y2)
        out_t[pl.ds(u * 2 * L, L)] = plsc.bitcast(y3, jnp.int32)
        out_t[pl.ds(u * 2 * L + L, L)] = bti

    def prefetch(qi, u):
        qw = w * QPW + lax.rem(qi, QPW)
        return pltpu.async_copy(
            cmb_hbm.at[pl.ds(qw * CMBW, CMBW)], cmb_ts[u], csems[u])

    def wait_prefetch(u):
        pltpu.make_async_copy(
            cmb_hbm.at[pl.ds(0, CMBW)], cmb_ts[u], csems[u]).wait()

    def per_body(p, carry):
        q0 = w * QPW + NQI * p
        gathers = []
        for u in range(NQI):
            wait_prefetch(u)
            sel_ts[u][...] = phase_a(cmb_ts[u])
            gathers.append(
                pltpu.async_copy(yg_hbm.at[sel_ts[u]], rows_ts[u], gsems[u]))
        for u in range(NQI):
            gathers[u].wait()
            btd, bti = phase_b(cmb_ts[u], sel_ts[u], rows_ts[u])
            emit(u, btd, bti)
            prefetch(NQI * p + u + NQI, u)
        pltpu.sync_copy(out_t, out_hbm.at[pl.ds(q0 * 2 * L, NQI * 2 * L)])
        return carry

    for u in range(NQI):
        prefetch(u, u)
    lax.fori_loop(0, QPW // NQI, per_body, 0)
    for u in range(NQI):
        wait_prefetch(u)


def kernel(x, y, k, n_splits):
    del k, n_splits
    yp = jnp.concatenate(
        [y, jnp.full((NP - N, DIM), PAD_VAL, jnp.float32)], axis=0)
    gm = pl.pallas_call(
        _gm_body,
        grid=(NP // CB, Q // QB),
        in_specs=[
            pl.BlockSpec((CB, DIM), lambda i, j: (i, 0)),
            pl.BlockSpec((DIM, QB), lambda i, j: (0, j)),
        ],
        out_specs=pl.BlockSpec((QB, CB // G), lambda i, j: (j, i)),
        out_shape=jax.ShapeDtypeStruct((Q, NG), jnp.float32),
    )(yp, x.T)
    yg = yp.reshape(NG, G * DIM)
    xbB = jnp.broadcast_to(x.reshape(Q * DIM, 1), (Q * DIM, L))
    cmb = jnp.concatenate([gm, xbB.reshape(Q, DIM * L)], axis=1)

    sc_call = pl.kernel(
        _sc_body,
        out_type=jax.ShapeDtypeStruct((Q * 2 * L,), jnp.int32),
        mesh=plsc.VectorSubcoreMesh(core_axis_name="c", subcore_axis_name="s"),
        compiler_params=pltpu.CompilerParams(needs_layout_passes=False),
        scratch_types=[
            [pltpu.VMEM((CMBW,), jnp.float32) for _ in range(NQI)],
            [pltpu.VMEM((NSEL,), jnp.int32) for _ in range(NQI)],
            [pltpu.VMEM((NSEL, G * DIM), jnp.float32) for _ in range(NQI)],
            pltpu.VMEM((NQI * 2 * L,), jnp.int32),
            [pltpu.SemaphoreType.DMA for _ in range(NQI)],
            [pltpu.SemaphoreType.DMA for _ in range(NQI)],
        ],
    )
    out = sc_call(cmb.reshape(-1), yg).reshape(Q, 2 * L)
    ds = lax.bitcast_convert_type(out[:, :K], jnp.float32)
    di = out[:, L:L + K]
    return ds, di[..., None]
